# baseline (speedup 1.0000x reference)
import jax
import jax.numpy as jnp
from jax import lax
from jax.experimental import pallas as pl
from jax.experimental.pallas import tpu as pltpu
from jax.experimental.pallas import tpu_sc as plsc

NC = 2
NS = 16
NW = NC * NS
L = 16

E = 320000
D = 128
DW = D // 2
EPW = E // NW
C = 80
NCHUNK = EPW // C
NGRP = C // L


def _sc_body(zu_hbm, zi_hbm, eidx_hbm, out_hbm,
             uidx, vidx, urows0, urows1, urows2, urows3,
             vrows0, vrows1, vrows2, vrows3, outv, pbuf,
             sem_u0, sem_u1, sem_u2, sem_u3,
             sem_v0, sem_v1, sem_v2, sem_v3):
    wid = lax.axis_index("s") * NC + lax.axis_index("c")
    base = wid * EPW

    pltpu.sync_copy(eidx_hbm.at[0, pl.ds(base, EPW)], uidx)
    pltpu.sync_copy(eidx_hbm.at[1, pl.ds(base, EPW)], vidx)

    ubufs = (urows0, urows1, urows2, urows3)
    vbufs = (vrows0, vrows1, vrows2, vrows3)
    usems = (sem_u0, sem_u1, sem_u2, sem_u3)
    vsems = (sem_v0, sem_v1, sem_v2, sem_v3)

    lane = lax.broadcasted_iota(jnp.int32, (L,), 0)
    one = jnp.float32(1.0)

    def start(cb, b):
        pltpu.async_copy(zu_hbm.at[uidx.at[pl.ds(cb * C, C)]], ubufs[b], usems[b])
        pltpu.async_copy(zi_hbm.at[vidx.at[pl.ds(cb * C, C)]], vbufs[b], vsems[b])

    def wait(cb, b):
        pltpu.make_async_copy(
            zu_hbm.at[uidx.at[pl.ds(cb * C, C)]], ubufs[b], usems[b]).wait()
        pltpu.make_async_copy(
            zi_hbm.at[vidx.at[pl.ds(cb * C, C)]], vbufs[b], vsems[b]).wait()

    lane17 = lane * 17

    def compute(cb, b):
        ur, vr = ubufs[b], vbufs[b]
        for e in range(C):
            acc0 = jnp.zeros((L,), jnp.float32)
            acc1 = jnp.zeros((L,), jnp.float32)
            for j in range(DW // L):
                u = plsc.bitcast(ur[e, pl.ds(j * L, L)], jnp.bfloat16)
                v = plsc.bitcast(vr[e, pl.ds(j * L, L)], jnp.bfloat16)
                p0, p1 = plsc.unpack(u * v, format=plsc.PackFormat.INTERLEAVED)
                acc0 = acc0 + p0
                acc1 = acc1 + p1
            pbuf[pl.ds(17 * e, L)] = acc0 + acc1
        for gg in range(NGRP):
            cols = [plsc.load_gather(pbuf, [lane17 + (gg * L * 17 + l)])
                    for l in range(L)]
            while len(cols) > 1:
                cols = [a + b for a, b in zip(cols[::2], cols[1::2])]
            tot = cols[0]
            s = one / (one + jnp.exp(-tot))
            outv[pl.ds(cb * C + gg * L, L)] = s

    for c0 in range(4):
        start(c0, c0)

    def body(g, carry):
        for b in range(4):
            cb = 4 * g + b
            wait(cb, b)
            compute(cb, b)
            nxt = cb + 4

            @pl.when(nxt < NCHUNK)
            def _prefetch():
                start(nxt, b)
        return carry

    lax.fori_loop(0, NCHUNK // 4, body, 0)
    wait(NCHUNK - 1, 0)
    compute(NCHUNK - 1, 0)

    pltpu.sync_copy(outv, out_hbm.at[pl.ds(base, EPW)])


@jax.jit
def _decode(z_user, z_item, edge_index):
    mesh = plsc.VectorSubcoreMesh(core_axis_name="c", subcore_axis_name="s")
    return pl.kernel(
        _sc_body,
        out_type=jax.ShapeDtypeStruct((E,), jnp.float32),
        mesh=mesh,
        compiler_params=pltpu.CompilerParams(needs_layout_passes=False, use_tc_tiling_on_sc=False),
        scratch_types=[
            pltpu.VMEM((EPW,), jnp.int32),
            pltpu.VMEM((EPW,), jnp.int32),
            pltpu.VMEM((C, DW), jnp.int32),
            pltpu.VMEM((C, DW), jnp.int32),
            pltpu.VMEM((C, DW), jnp.int32),
            pltpu.VMEM((C, DW), jnp.int32),
            pltpu.VMEM((C, DW), jnp.int32),
            pltpu.VMEM((C, DW), jnp.int32),
            pltpu.VMEM((C, DW), jnp.int32),
            pltpu.VMEM((C, DW), jnp.int32),
            pltpu.VMEM((EPW,), jnp.float32),
            pltpu.VMEM((C * 17,), jnp.float32),
            pltpu.SemaphoreType.DMA,
            pltpu.SemaphoreType.DMA,
            pltpu.SemaphoreType.DMA,
            pltpu.SemaphoreType.DMA,
            pltpu.SemaphoreType.DMA,
            pltpu.SemaphoreType.DMA,
            pltpu.SemaphoreType.DMA,
            pltpu.SemaphoreType.DMA,
        ],
    )(z_user, z_item, edge_index)


def _pack_rows(z):
    a = jax.lax.bitcast_convert_type(
        z[:, :DW].astype(jnp.bfloat16).astype(jnp.float32), jnp.uint32)
    b = jax.lax.bitcast_convert_type(
        z[:, DW:].astype(jnp.bfloat16).astype(jnp.float32), jnp.uint32)
    return jax.lax.bitcast_convert_type(a | (b >> 16), jnp.int32)


def kernel(z_user, z_item, edge_index):
    return _decode(_pack_rows(z_user), _pack_rows(z_item), edge_index)


# speedup vs baseline: 1.1431x; 1.1431x over previous
import jax
import jax.numpy as jnp
from jax import lax
from jax.experimental import pallas as pl
from jax.experimental.pallas import tpu as pltpu
from jax.experimental.pallas import tpu_sc as plsc

NC = 2
NS = 16
NW = NC * NS
L = 16

E = 320000
N_ROWS = 10000
D = 128
DW = D // 2
EPW = E // NW
C = 80
NCHUNK = EPW // C
NGRP = C // L


def _sc_body(zu_hbm, zi_hbm, eidx_hbm, out_hbm,
             uidx, vidx, urows0, urows1, vrows0, vrows1, outv, pbuf,
             zu_sh, sem_u0, sem_u1, sem_v0, sem_v1):
    wid = lax.axis_index("s") * NC + lax.axis_index("c")
    base = wid * EPW

    pltpu.sync_copy(eidx_hbm.at[0, pl.ds(base, EPW)], uidx)
    pltpu.sync_copy(eidx_hbm.at[1, pl.ds(base, EPW)], vidx)

    sid = lax.axis_index("s")
    rpt = N_ROWS // NS
    pltpu.sync_copy(zu_hbm.at[pl.ds(sid * rpt, rpt)], zu_sh.at[pl.ds(sid * rpt, rpt)])
    plsc.subcore_barrier()

    ubufs = (urows0, urows1)
    vbufs = (vrows0, vrows1)
    usems = (sem_u0, sem_u1)
    vsems = (sem_v0, sem_v1)

    lane = lax.broadcasted_iota(jnp.int32, (L,), 0)
    one = jnp.float32(1.0)

    def start(cb, b):
        pltpu.async_copy(zu_sh.at[uidx.at[pl.ds(cb * C, C)]], ubufs[b], usems[b])
        pltpu.async_copy(zi_hbm.at[vidx.at[pl.ds(cb * C, C)]], vbufs[b], vsems[b])

    def wait(cb, b):
        pltpu.make_async_copy(
            zu_sh.at[uidx.at[pl.ds(cb * C, C)]], ubufs[b], usems[b]).wait()
        pltpu.make_async_copy(
            zi_hbm.at[vidx.at[pl.ds(cb * C, C)]], vbufs[b], vsems[b]).wait()

    lane17 = lane * 17

    def compute(cb, b):
        ur, vr = ubufs[b], vbufs[b]
        for e in range(C):
            acc0 = jnp.zeros((L,), jnp.float32)
            acc1 = jnp.zeros((L,), jnp.float32)
            for j in range(DW // L):
                u = plsc.bitcast(ur[e, pl.ds(j * L, L)], jnp.bfloat16)
                v = plsc.bitcast(vr[e, pl.ds(j * L, L)], jnp.bfloat16)
                p0, p1 = plsc.unpack(u * v, format=plsc.PackFormat.INTERLEAVED)
                acc0 = acc0 + p0
                acc1 = acc1 + p1
            pbuf[pl.ds(17 * e, L)] = acc0 + acc1
        for gg in range(NGRP):
            cols = [plsc.load_gather(pbuf, [lane17 + (gg * L * 17 + l)])
                    for l in range(L)]
            while len(cols) > 1:
                cols = [a + b for a, b in zip(cols[::2], cols[1::2])]
            tot = cols[0]
            s = one / (one + jnp.exp(-tot))
            outv[pl.ds(cb * C + gg * L, L)] = s

    start(0, 0)
    start(1, 1)

    def body(g, carry):
        for b in range(2):
            cb = 2 * g + b
            wait(cb, b)
            compute(cb, b)
            nxt = cb + 2

            @pl.when(nxt < NCHUNK)
            def _prefetch():
                start(nxt, b)
        return carry

    lax.fori_loop(0, NCHUNK // 2, body, 0)
    wait(NCHUNK - 1, 0)
    compute(NCHUNK - 1, 0)

    pltpu.sync_copy(outv, out_hbm.at[pl.ds(base, EPW)])


@jax.jit
def _decode(z_user, z_item, edge_index):
    mesh = plsc.VectorSubcoreMesh(core_axis_name="c", subcore_axis_name="s")
    return pl.kernel(
        _sc_body,
        out_type=jax.ShapeDtypeStruct((E,), jnp.float32),
        mesh=mesh,
        compiler_params=pltpu.CompilerParams(needs_layout_passes=False, use_tc_tiling_on_sc=False),
        scratch_types=[
            pltpu.VMEM((EPW,), jnp.int32),
            pltpu.VMEM((EPW,), jnp.int32),
            pltpu.VMEM((C, DW), jnp.int32),
            pltpu.VMEM((C, DW), jnp.int32),
            pltpu.VMEM((C, DW), jnp.int32),
            pltpu.VMEM((C, DW), jnp.int32),
            pltpu.VMEM((EPW,), jnp.float32),
            pltpu.VMEM((C * 17,), jnp.float32),
            pltpu.VMEM_SHARED((N_ROWS, DW), jnp.int32),
            pltpu.SemaphoreType.DMA,
            pltpu.SemaphoreType.DMA,
            pltpu.SemaphoreType.DMA,
            pltpu.SemaphoreType.DMA,
        ],
    )(z_user, z_item, edge_index)


def _pack_rows(z):
    a = jax.lax.bitcast_convert_type(
        z[:, :DW].astype(jnp.bfloat16).astype(jnp.float32), jnp.uint32)
    b = jax.lax.bitcast_convert_type(
        z[:, DW:].astype(jnp.bfloat16).astype(jnp.float32), jnp.uint32)
    return jax.lax.bitcast_convert_type(a | (b >> 16), jnp.int32)


def kernel(z_user, z_item, edge_index):
    return _decode(_pack_rows(z_user), _pack_rows(z_item), edge_index)
